# Initial kernel scaffold; baseline (speedup 1.0000x reference)
import jax
import jax.numpy as jnp
from jax.experimental import pallas as pl

DIM_A = 128
DIM_V = 32
CHAN = 64
HDIM = 64
R0 = 5.0
N_NODES = 10000
N_TYPES = 32
GROUPS = 8

_INTERPRET = False


def _lrelu(x):
    return jnp.where(x >= 0, x, 0.1 * x)


def _mlp(p, x):
    h = _lrelu(x @ p['W1'] + p['b1'])
    h = _lrelu(h @ p['W2'] + p['b2'])
    return h @ p['W3']


def _radial_encode(r, n, rmax):
    sq = jnp.sum(r ** 2, axis=-1, keepdims=True) / rmax ** 2
    coeffs = jnp.pi * 2 ** -0.5 * jnp.exp(float(n) ** -0.7 * jnp.arange(n, dtype=jnp.float32))
    hyper = jnp.sqrt(1.0 + sq)
    phase = coeffs * hyper
    return jnp.sin(phase) * (0.5 ** 0.5 - hyper)


def _tens_sigmoid1(x):
    return x / jnp.sqrt(1.0 + jnp.sum(x ** 2, axis=-1, keepdims=True))


def _prologue_kernel(xa_ref, idx_ref, t_ref, res_ref, ew_ref, eb_ref, out_ref):
    t = t_ref[0, 0]
    freqs = jnp.exp(-jax.lax.broadcasted_iota(jnp.float32, (1, HDIM), 1) / (HDIM - 1))
    phases = t * freqs
    raw = jnp.concatenate([jnp.sin(phases), jnp.cos(phases)], axis=1)
    te = raw @ ew_ref[...] + eb_ref[...]
    idx = idx_ref[...]
    onehot = (idx == jax.lax.broadcasted_iota(jnp.int32, (idx.shape[0], N_TYPES), 1)
              ).astype(jnp.float32)
    out_ref[...] = xa_ref[...] + te + onehot @ res_ref[...]


def _prologue(x_a, atom_idx, t, res_embed, embed_W, embed_b):
    n = x_a.shape[0]
    return pl.pallas_call(
        _prologue_kernel,
        out_shape=jax.ShapeDtypeStruct((n, DIM_A), jnp.float32),
        interpret=_INTERPRET,
    )(x_a, atom_idx.reshape(n, 1), t.reshape(1, 1), res_embed,
      embed_W, embed_b.reshape(1, DIM_A))


def _tl1(x, W):
    return jnp.einsum('...di,do->...oi', x, W)


def _disptens(p, src, dst, r_ij, x_a):
    rad = _mlp(p['mlp'], _radial_encode(r_ij, DIM_A, R0) + x_a[dst])
    rs = _tens_sigmoid1(r_ij * (7.0 / R0))
    rv = rad @ p['readout_v']
    A_a = jax.ops.segment_sum(rad, src, num_segments=N_NODES)
    A_v = jax.ops.segment_sum(rv[..., None] * rs[..., None, :], src, num_segments=N_NODES)
    return A_a, A_v


def _messages(p, src, dst, r_ij, x_a, x_v):
    rad = _radial_encode(r_ij, DIM_A, R0) @ p['enc_W'] + p['enc_b']
    rs = _tens_sigmoid1(r_ij * (7.0 / R0))
    phi_v = rad[..., None] * rs[..., None, :]
    l_a = (x_a @ p['L0'])[dst]
    l_v = _tl1(x_v, p['L1'])[dst]
    psi_a = (l_a * rad) @ p['Y000'] + jnp.sum(l_v * phi_v, -1) @ p['Y110']
    psi_v = (_tl1(l_a[..., None] * phi_v, p['Y011'])
             + _tl1(l_v * rad[..., None], p['Y101'])
             + _tl1(jnp.cross(l_v, phi_v), p['Y111']))
    psi_a = psi_a + _mlp(p['mlp'], psi_a)
    B_a = jax.ops.segment_sum(psi_a, src, num_segments=N_NODES)
    B_v = jax.ops.segment_sum(psi_v, src, num_segments=N_NODES)
    return 0.1 * B_a, 0.1 * B_v


def _group_norm0(x, g, b):
    sh = x.shape
    d = sh[-1]
    xg = x.reshape(sh[:-1] + (GROUPS, d // GROUPS))
    mu = xg.mean(-1, keepdims=True)
    var = xg.var(-1, keepdims=True)
    xg = (xg - mu) / jnp.sqrt(var + 1e-5)
    return xg.reshape(sh) * g + b


def _group_norm1(x, g):
    sh = x.shape
    d = sh[-2]
    xg = x.reshape(sh[:-2] + (GROUPS, d // GROUPS, 3))
    rms = jnp.sqrt(jnp.mean(jnp.sum(xg ** 2, -1), -1, keepdims=True) + 1e-5)
    xg = xg / rms[..., None]
    return xg.reshape(sh) * g[:, None]


def _self_tens_prods(p, x_a, x_v):
    l0 = x_a @ p['L0']
    r0 = x_a @ p['R0']
    l1 = _tl1(x_v, p['L1'])
    r1 = _tl1(x_v, p['R1'])
    y_a = (l0 * r0) @ p['Y000'] + jnp.sum(l1 * r1, -1) @ p['Y110']
    y_a = y_a + _mlp(p['mlp'], y_a)
    l0 = l0 + y_a @ p['Ul']
    r0 = r0 + y_a @ p['Ur']
    y_v = (_tl1(l0[..., None] * r1, p['Y011'])
           + _tl1(l1 * r0[..., None], p['Y101'])
           + _tl1(jnp.cross(l1, r1), p['Y111']))
    return _group_norm0(y_a, p['gn_a_g'], p['gn_a_b']), _group_norm1(y_v, p['gn_v_g'])


def kernel(pos_0, pos_1, x_a, x_v, t, atom_idx, edge_index, params):
    src = edge_index[0]
    dst = edge_index[1]
    xa = _prologue(x_a[0], atom_idx[0].astype(jnp.int32), t,
                   params['res_embed'], params['embed_t']['W'],
                   params['embed_t']['b'])
    xv = x_v[0] + (pos_1 - pos_0)[0, :, None, :] * params['pos_embed'][0][:, None]
    r0_ij = pos_0[0][dst] - pos_0[0][src]
    r1_ij = pos_1[0][dst] - pos_1[0][src]
    A_a0, A_v0 = _disptens(params['disptens_0'], src, dst, r0_ij, xa)
    A_a1, A_v1 = _disptens(params['disptens_1'], src, dst, r1_ij, xa)
    xa = xa + A_a0 + A_a1
    xv = xv + A_v0 + A_v1
    y_a, y_v = _self_tens_prods(params['stp'], xa, xv)
    xa = xa + y_a
    xv = xv + y_v
    B_a0, B_v0 = _messages(params['msg_0'], src, dst, r0_ij, xa, xv)
    B_a1, B_v1 = _messages(params['msg_1'], src, dst, r1_ij, xa, xv)
    xa = xa + B_a0 + B_a1
    xv = xv + B_v0 + B_v1
    return xa[None], xv[None]


# speedup vs baseline: 1.0891x; 1.0891x over previous
import jax
import jax.numpy as jnp
from jax.experimental import pallas as pl

DIM_A = 128
DIM_V = 32
CHAN = 64
HDIM = 64
R0 = 5.0
N_NODES = 10000
N_TYPES = 32
GROUPS = 8

_INTERPRET = False


def _lrelu(x):
    return jnp.where(x >= 0, x, 0.1 * x)


def _mlp(p, x):
    h = _lrelu(x @ p['W1'] + p['b1'])
    h = _lrelu(h @ p['W2'] + p['b2'])
    return h @ p['W3']


def _radial_encode(r, n, rmax):
    sq = jnp.sum(r ** 2, axis=-1, keepdims=True) / rmax ** 2
    coeffs = jnp.pi * 2 ** -0.5 * jnp.exp(float(n) ** -0.7 * jnp.arange(n, dtype=jnp.float32))
    hyper = jnp.sqrt(1.0 + sq)
    phase = coeffs * hyper
    return jnp.sin(phase) * (0.5 ** 0.5 - hyper)


def _tens_sigmoid1(x):
    return x / jnp.sqrt(1.0 + jnp.sum(x ** 2, axis=-1, keepdims=True))


def _prologue_kernel(xa_ref, oh_ref, raw_ref, res_ref, ew_ref, eb_ref, out_ref):
    te = raw_ref[...] @ ew_ref[...] + eb_ref[...]
    out_ref[...] = xa_ref[...] + te + oh_ref[...] @ res_ref[...]


def _prologue(x_a, atom_idx, t, res_embed, embed_W, embed_b):
    n = x_a.shape[0]
    onehot = (atom_idx[:, None] == jnp.arange(N_TYPES)[None, :]).astype(jnp.float32)
    freqs = jnp.exp(-jnp.arange(HDIM, dtype=jnp.float32) / (HDIM - 1))
    phases = t[:, None] * freqs[None, :]
    raw = jnp.concatenate([jnp.sin(phases), jnp.cos(phases)], axis=1)
    return pl.pallas_call(
        _prologue_kernel,
        out_shape=jax.ShapeDtypeStruct((n, DIM_A), jnp.float32),
        interpret=_INTERPRET,
    )(x_a, onehot, raw, res_embed, embed_W, embed_b.reshape(1, DIM_A))


def _tl1(x, W):
    return jnp.einsum('...di,do->...oi', x, W)


def _disptens(p, src, dst, r_ij, x_a):
    rad = _mlp(p['mlp'], _radial_encode(r_ij, DIM_A, R0) + x_a[dst])
    rs = _tens_sigmoid1(r_ij * (7.0 / R0))
    rv = rad @ p['readout_v']
    A_a = jax.ops.segment_sum(rad, src, num_segments=N_NODES)
    A_v = jax.ops.segment_sum(rv[..., None] * rs[..., None, :], src, num_segments=N_NODES)
    return A_a, A_v


def _messages(p, src, dst, r_ij, x_a, x_v):
    rad = _radial_encode(r_ij, DIM_A, R0) @ p['enc_W'] + p['enc_b']
    rs = _tens_sigmoid1(r_ij * (7.0 / R0))
    l_a = (x_a @ p['L0'])[dst]
    lv_n = _tl1(x_v, p['L1'])
    lv0 = lv_n[:, :, 0][dst]
    lv1 = lv_n[:, :, 1][dst]
    lv2 = lv_n[:, :, 2][dst]
    rs0 = rs[:, 0:1]
    rs1 = rs[:, 1:2]
    rs2 = rs[:, 2:3]
    lrs = lv0 * rs0 + lv1 * rs1 + lv2 * rs2
    psi_a = (l_a * rad) @ p['Y000'] + (rad * lrs) @ p['Y110']
    psi_a = psi_a + _mlp(p['mlp'], psi_a)
    c0 = lv1 * rs2 - lv2 * rs1
    c1 = lv2 * rs0 - lv0 * rs2
    c2 = lv0 * rs1 - lv1 * rs0
    pv0 = (l_a * rad * rs0) @ p['Y011'] + (rad * lv0) @ p['Y101'] + (rad * c0) @ p['Y111']
    pv1 = (l_a * rad * rs1) @ p['Y011'] + (rad * lv1) @ p['Y101'] + (rad * c1) @ p['Y111']
    pv2 = (l_a * rad * rs2) @ p['Y011'] + (rad * lv2) @ p['Y101'] + (rad * c2) @ p['Y111']
    psi_v = jnp.stack([pv0, pv1, pv2], axis=-1)
    B_a = jax.ops.segment_sum(psi_a, src, num_segments=N_NODES)
    B_v = jax.ops.segment_sum(psi_v, src, num_segments=N_NODES)
    return 0.1 * B_a, 0.1 * B_v


def _group_norm0(x, g, b):
    sh = x.shape
    d = sh[-1]
    xg = x.reshape(sh[:-1] + (GROUPS, d // GROUPS))
    mu = xg.mean(-1, keepdims=True)
    var = xg.var(-1, keepdims=True)
    xg = (xg - mu) / jnp.sqrt(var + 1e-5)
    return xg.reshape(sh) * g + b


def _group_norm1(x, g):
    sh = x.shape
    d = sh[-2]
    xg = x.reshape(sh[:-2] + (GROUPS, d // GROUPS, 3))
    rms = jnp.sqrt(jnp.mean(jnp.sum(xg ** 2, -1), -1, keepdims=True) + 1e-5)
    xg = xg / rms[..., None]
    return xg.reshape(sh) * g[:, None]


def _self_tens_prods(p, x_a, x_v):
    l0 = x_a @ p['L0']
    r0 = x_a @ p['R0']
    l1 = _tl1(x_v, p['L1'])
    r1 = _tl1(x_v, p['R1'])
    y_a = (l0 * r0) @ p['Y000'] + jnp.sum(l1 * r1, -1) @ p['Y110']
    y_a = y_a + _mlp(p['mlp'], y_a)
    l0 = l0 + y_a @ p['Ul']
    r0 = r0 + y_a @ p['Ur']
    y_v = (_tl1(l0[..., None] * r1, p['Y011'])
           + _tl1(l1 * r0[..., None], p['Y101'])
           + _tl1(jnp.cross(l1, r1), p['Y111']))
    return _group_norm0(y_a, p['gn_a_g'], p['gn_a_b']), _group_norm1(y_v, p['gn_v_g'])


def kernel(pos_0, pos_1, x_a, x_v, t, atom_idx, edge_index, params):
    src = edge_index[0]
    dst = edge_index[1]
    xa = _prologue(x_a[0], atom_idx[0].astype(jnp.int32), t,
                   params['res_embed'], params['embed_t']['W'],
                   params['embed_t']['b'])
    xv = x_v[0] + (pos_1 - pos_0)[0, :, None, :] * params['pos_embed'][0][:, None]
    r0_ij = pos_0[0][dst] - pos_0[0][src]
    r1_ij = pos_1[0][dst] - pos_1[0][src]
    A_a0, A_v0 = _disptens(params['disptens_0'], src, dst, r0_ij, xa)
    A_a1, A_v1 = _disptens(params['disptens_1'], src, dst, r1_ij, xa)
    xa = xa + A_a0 + A_a1
    xv = xv + A_v0 + A_v1
    y_a, y_v = _self_tens_prods(params['stp'], xa, xv)
    xa = xa + y_a
    xv = xv + y_v
    B_a0, B_v0 = _messages(params['msg_0'], src, dst, r0_ij, xa, xv)
    B_a1, B_v1 = _messages(params['msg_1'], src, dst, r1_ij, xa, xv)
    xa = xa + B_a0 + B_a1
    xv = xv + B_v0 + B_v1
    return xa[None], xv[None]


# speedup vs baseline: 5.2551x; 4.8252x over previous
import jax
import jax.numpy as jnp
from jax.experimental import pallas as pl

DIM_A = 128
DIM_V = 32
CHAN = 64
HDIM = 64
R0 = 5.0
N_NODES = 10000
N_TYPES = 32
GROUPS = 8

_INTERPRET = False


def _lrelu(x):
    return jnp.where(x >= 0, x, 0.1 * x)


def _mlp(p, x):
    h = _lrelu(x @ p['W1'] + p['b1'])
    h = _lrelu(h @ p['W2'] + p['b2'])
    return h @ p['W3']


def _radial_encode(r, n, rmax):
    sq = jnp.sum(r ** 2, axis=-1, keepdims=True) / rmax ** 2
    coeffs = jnp.pi * 2 ** -0.5 * jnp.exp(float(n) ** -0.7 * jnp.arange(n, dtype=jnp.float32))
    hyper = jnp.sqrt(1.0 + sq)
    phase = coeffs * hyper
    return jnp.sin(phase) * (0.5 ** 0.5 - hyper)


def _tens_sigmoid1(x):
    return x / jnp.sqrt(1.0 + jnp.sum(x ** 2, axis=-1, keepdims=True))


def _prologue_kernel(xa_ref, oh_ref, raw_ref, res_ref, ew_ref, eb_ref, out_ref):
    te = raw_ref[...] @ ew_ref[...] + eb_ref[...]
    out_ref[...] = xa_ref[...] + te + oh_ref[...] @ res_ref[...]


def _prologue(x_a, atom_idx, t, res_embed, embed_W, embed_b):
    n = x_a.shape[0]
    onehot = (atom_idx[:, None] == jnp.arange(N_TYPES)[None, :]).astype(jnp.float32)
    freqs = jnp.exp(-jnp.arange(HDIM, dtype=jnp.float32) / (HDIM - 1))
    phases = t[:, None] * freqs[None, :]
    raw = jnp.concatenate([jnp.sin(phases), jnp.cos(phases)], axis=1)
    return pl.pallas_call(
        _prologue_kernel,
        out_shape=jax.ShapeDtypeStruct((n, DIM_A), jnp.float32),
        interpret=_INTERPRET,
    )(x_a, onehot, raw, res_embed, embed_W, embed_b.reshape(1, DIM_A))


def _tl1(x, W):
    return jnp.einsum('...di,do->...oi', x, W)


def _disptens_payload(p, dst, r_ij, x_a):
    rad = _mlp(p['mlp'], _radial_encode(r_ij, DIM_A, R0) + x_a[dst])
    rs = _tens_sigmoid1(r_ij * (7.0 / R0))
    rv = rad @ p['readout_v']
    return jnp.concatenate(
        [rad, rv * rs[:, 0:1], rv * rs[:, 1:2], rv * rs[:, 2:3]], axis=1)


def _messages_payload(p, dst, r_ij, x_a, x_v):
    rad = _radial_encode(r_ij, DIM_A, R0) @ p['enc_W'] + p['enc_b']
    rs = _tens_sigmoid1(r_ij * (7.0 / R0))
    l_a = (x_a @ p['L0'])[dst]
    lv_n = _tl1(x_v, p['L1'])
    lv0 = lv_n[:, :, 0][dst]
    lv1 = lv_n[:, :, 1][dst]
    lv2 = lv_n[:, :, 2][dst]
    rs0 = rs[:, 0:1]
    rs1 = rs[:, 1:2]
    rs2 = rs[:, 2:3]
    lrs = lv0 * rs0 + lv1 * rs1 + lv2 * rs2
    psi_a = (l_a * rad) @ p['Y000'] + (rad * lrs) @ p['Y110']
    psi_a = psi_a + _mlp(p['mlp'], psi_a)
    c0 = lv1 * rs2 - lv2 * rs1
    c1 = lv2 * rs0 - lv0 * rs2
    c2 = lv0 * rs1 - lv1 * rs0
    pv0 = (l_a * rad * rs0) @ p['Y011'] + (rad * lv0) @ p['Y101'] + (rad * c0) @ p['Y111']
    pv1 = (l_a * rad * rs1) @ p['Y011'] + (rad * lv1) @ p['Y101'] + (rad * c1) @ p['Y111']
    pv2 = (l_a * rad * rs2) @ p['Y011'] + (rad * lv2) @ p['Y101'] + (rad * c2) @ p['Y111']
    return jnp.concatenate([psi_a, pv0, pv1, pv2], axis=1)


def _group_norm0(x, g, b):
    sh = x.shape
    d = sh[-1]
    xg = x.reshape(sh[:-1] + (GROUPS, d // GROUPS))
    mu = xg.mean(-1, keepdims=True)
    var = xg.var(-1, keepdims=True)
    xg = (xg - mu) / jnp.sqrt(var + 1e-5)
    return xg.reshape(sh) * g + b


def _group_norm1(x, g):
    sh = x.shape
    d = sh[-2]
    xg = x.reshape(sh[:-2] + (GROUPS, d // GROUPS, 3))
    rms = jnp.sqrt(jnp.mean(jnp.sum(xg ** 2, -1), -1, keepdims=True) + 1e-5)
    xg = xg / rms[..., None]
    return xg.reshape(sh) * g[:, None]


def _self_tens_prods(p, x_a, x_v):
    l0 = x_a @ p['L0']
    r0 = x_a @ p['R0']
    l1 = _tl1(x_v, p['L1'])
    r1 = _tl1(x_v, p['R1'])
    y_a = (l0 * r0) @ p['Y000'] + jnp.sum(l1 * r1, -1) @ p['Y110']
    y_a = y_a + _mlp(p['mlp'], y_a)
    l0 = l0 + y_a @ p['Ul']
    r0 = r0 + y_a @ p['Ur']
    y_v = (_tl1(l0[..., None] * r1, p['Y011'])
           + _tl1(l1 * r0[..., None], p['Y101'])
           + _tl1(jnp.cross(l1, r1), p['Y111']))
    return _group_norm0(y_a, p['gn_a_g'], p['gn_a_b']), _group_norm1(y_v, p['gn_v_g'])


def kernel(pos_0, pos_1, x_a, x_v, t, atom_idx, edge_index, params):
    src = edge_index[0]
    dst = edge_index[1]
    xa = _prologue(x_a[0], atom_idx[0].astype(jnp.int32), t,
                   params['res_embed'], params['embed_t']['W'],
                   params['embed_t']['b'])
    xv = x_v[0] + (pos_1 - pos_0)[0, :, None, :] * params['pos_embed'][0][:, None]
    r0_ij = pos_0[0][dst] - pos_0[0][src]
    r1_ij = pos_1[0][dst] - pos_1[0][src]
    n = xa.shape[0]

    pay = jnp.concatenate([_disptens_payload(params['disptens_0'], dst, r0_ij, xa),
                           _disptens_payload(params['disptens_1'], dst, r1_ij, xa)], axis=1)
    A = jax.ops.segment_sum(pay, src, num_segments=N_NODES)
    xa = xa + A[:, :DIM_A] + A[:, 224:224 + DIM_A]
    A_v = (A[:, DIM_A:224].reshape(n, 3, DIM_V)
           + A[:, 224 + DIM_A:].reshape(n, 3, DIM_V)).transpose(0, 2, 1)
    xv = xv + A_v

    y_a, y_v = _self_tens_prods(params['stp'], xa, xv)
    xa = xa + y_a
    xv = xv + y_v

    pay = jnp.concatenate([_messages_payload(params['msg_0'], dst, r0_ij, xa, xv),
                           _messages_payload(params['msg_1'], dst, r1_ij, xa, xv)], axis=1)
    B = jax.ops.segment_sum(pay, src, num_segments=N_NODES)
    xa = xa + 0.1 * (B[:, :DIM_A] + B[:, 224:224 + DIM_A])
    B_v = (B[:, DIM_A:224].reshape(n, 3, DIM_V)
           + B[:, 224 + DIM_A:].reshape(n, 3, DIM_V)).transpose(0, 2, 1)
    xv = xv + 0.1 * B_v
    return xa[None], xv[None]


# speedup vs baseline: 7.5811x; 1.4426x over previous
import jax
import jax.numpy as jnp
from jax.experimental import pallas as pl

DIM_A = 128
DIM_V = 32
CHAN = 64
HDIM = 64
R0 = 5.0
N_NODES = 10000
N_TYPES = 32
GROUPS = 8

_INTERPRET = False


def _lrelu(x):
    return jnp.where(x >= 0, x, 0.1 * x)


def _mlp(p, x):
    h = _lrelu(x @ p['W1'] + p['b1'])
    h = _lrelu(h @ p['W2'] + p['b2'])
    return h @ p['W3']


def _radial_encode(r, n, rmax):
    sq = jnp.sum(r ** 2, axis=-1, keepdims=True) / rmax ** 2
    coeffs = jnp.pi * 2 ** -0.5 * jnp.exp(float(n) ** -0.7 * jnp.arange(n, dtype=jnp.float32))
    hyper = jnp.sqrt(1.0 + sq)
    phase = coeffs * hyper
    return jnp.sin(phase) * (0.5 ** 0.5 - hyper)


def _tens_sigmoid1(x):
    return x / jnp.sqrt(1.0 + jnp.sum(x ** 2, axis=-1, keepdims=True))


def _prologue_kernel(xa_ref, oh_ref, raw_ref, res_ref, ew_ref, eb_ref, out_ref):
    te = raw_ref[...] @ ew_ref[...] + eb_ref[...]
    out_ref[...] = xa_ref[...] + te + oh_ref[...] @ res_ref[...]


def _prologue(x_a, atom_idx, t, res_embed, embed_W, embed_b):
    n = x_a.shape[0]
    onehot = (atom_idx[:, None] == jnp.arange(N_TYPES)[None, :]).astype(jnp.float32)
    freqs = jnp.exp(-jnp.arange(HDIM, dtype=jnp.float32) / (HDIM - 1))
    phases = t[:, None] * freqs[None, :]
    raw = jnp.concatenate([jnp.sin(phases), jnp.cos(phases)], axis=1)
    return pl.pallas_call(
        _prologue_kernel,
        out_shape=jax.ShapeDtypeStruct((n, DIM_A), jnp.float32),
        interpret=_INTERPRET,
    )(x_a, onehot, raw, res_embed, embed_W, embed_b.reshape(1, DIM_A))


def _tl1(x, W):
    return jnp.einsum('...di,do->...oi', x, W)


_EBLK = 1000


def _enc_rs(r):
    sq = jnp.sum(r ** 2, axis=1, keepdims=True) / R0 ** 2
    ar = jax.lax.broadcasted_iota(jnp.int32, (1, DIM_A), 1).astype(jnp.float32)
    coeffs = jnp.pi * 2 ** -0.5 * jnp.exp(float(DIM_A) ** -0.7 * ar)
    hyper = jnp.sqrt(1.0 + sq)
    enc = jnp.sin(coeffs * hyper) * (0.5 ** 0.5 - hyper)
    rr = r * (7.0 / R0)
    rs = rr / jnp.sqrt(1.0 + jnp.sum(rr ** 2, axis=1, keepdims=True))
    return enc, rs


def _mlp_r(x, w1, b1, w2, b2, w3):
    h = _lrelu(x @ w1[...] + b1[...])
    h = _lrelu(h @ w2[...] + b2[...])
    return h @ w3[...]


def _disptens_edge_kernel(rp_ref, xad_ref, w10, b10, w20, b20, w30, ro0,
                          w11, b11, w21, b21, w31, ro1, out_ref):
    rp = rp_ref[...]
    xad = xad_ref[...]
    halves = []
    for (w1, b1, w2, b2, w3, ro, sl) in (
            (w10, b10, w20, b20, w30, ro0, slice(0, 3)),
            (w11, b11, w21, b21, w31, ro1, slice(3, 6))):
        enc, rs = _enc_rs(rp[:, sl])
        rad = _mlp_r(enc + xad, w1, b1, w2, b2, w3)
        rv = rad @ ro[...]
        halves.append(jnp.concatenate(
            [rad, rv * rs[:, 0:1], rv * rs[:, 1:2], rv * rs[:, 2:3]], axis=1))
    out_ref[...] = jnp.concatenate(halves, axis=1)


def _disptens_edges(rpack, xa_dst, p0, p1):
    e = rpack.shape[0]
    grid = e // _EBLK
    ws = []
    for p in (p0, p1):
        m = p['mlp']
        ws += [m['W1'], m['b1'].reshape(1, DIM_A), m['W2'], m['b2'].reshape(1, DIM_A),
               m['W3'], p['readout_v']]
    wspec = [pl.BlockSpec(w.shape, lambda i: (0,) * w.ndim) for w in ws]
    return pl.pallas_call(
        _disptens_edge_kernel,
        grid=(grid,),
        in_specs=[pl.BlockSpec((_EBLK, 6), lambda i: (i, 0)),
                  pl.BlockSpec((_EBLK, DIM_A), lambda i: (i, 0))] + wspec,
        out_specs=pl.BlockSpec((_EBLK, 448), lambda i: (i, 0)),
        out_shape=jax.ShapeDtypeStruct((e, 448), jnp.float32),
        interpret=_INTERPRET,
    )(rpack, xa_dst, *ws)


def _messages_edge_kernel(rp_ref, g_ref,
                          ew0, eb0, y000_0, y110_0, y011_0, y101_0, y111_0,
                          w10, b10, w20, b20, w30,
                          ew1, eb1, y000_1, y110_1, y011_1, y101_1, y111_1,
                          w11, b11, w21, b21, w31, out_ref):
    rp = rp_ref[...]
    g = g_ref[...]
    halves = []
    for (ew, eb, y000, y110, y011, y101, y111, w1, b1, w2, b2, w3, sl, go) in (
            (ew0, eb0, y000_0, y110_0, y011_0, y101_0, y111_0,
             w10, b10, w20, b20, w30, slice(0, 3), 0),
            (ew1, eb1, y000_1, y110_1, y011_1, y101_1, y111_1,
             w11, b11, w21, b21, w31, slice(3, 6), 256)):
        enc, rs = _enc_rs(rp[:, sl])
        rad = enc @ ew[...] + eb[...]
        l_a = g[:, go:go + CHAN]
        lv0 = g[:, go + CHAN:go + 2 * CHAN]
        lv1 = g[:, go + 2 * CHAN:go + 3 * CHAN]
        lv2 = g[:, go + 3 * CHAN:go + 4 * CHAN]
        rs0 = rs[:, 0:1]
        rs1 = rs[:, 1:2]
        rs2 = rs[:, 2:3]
        lrs = lv0 * rs0 + lv1 * rs1 + lv2 * rs2
        psi_a = (l_a * rad) @ y000[...] + (rad * lrs) @ y110[...]
        psi_a = psi_a + _mlp_r(psi_a, w1, b1, w2, b2, w3)
        c0 = lv1 * rs2 - lv2 * rs1
        c1 = lv2 * rs0 - lv0 * rs2
        c2 = lv0 * rs1 - lv1 * rs0
        pv0 = (l_a * rad * rs0) @ y011[...] + (rad * lv0) @ y101[...] + (rad * c0) @ y111[...]
        pv1 = (l_a * rad * rs1) @ y011[...] + (rad * lv1) @ y101[...] + (rad * c1) @ y111[...]
        pv2 = (l_a * rad * rs2) @ y011[...] + (rad * lv2) @ y101[...] + (rad * c2) @ y111[...]
        halves.append(jnp.concatenate([psi_a, pv0, pv1, pv2], axis=1))
    out_ref[...] = jnp.concatenate(halves, axis=1)


def _messages_edges(rpack, gathered, p0, p1):
    e = rpack.shape[0]
    grid = e // _EBLK
    ws = []
    for p in (p0, p1):
        m = p['mlp']
        ws += [p['enc_W'], p['enc_b'].reshape(1, CHAN), p['Y000'], p['Y110'],
               p['Y011'], p['Y101'], p['Y111'],
               m['W1'], m['b1'].reshape(1, DIM_A), m['W2'], m['b2'].reshape(1, DIM_A),
               m['W3']]
    wspec = [pl.BlockSpec(w.shape, lambda i: (0,) * w.ndim) for w in ws]
    return pl.pallas_call(
        _messages_edge_kernel,
        grid=(grid,),
        in_specs=[pl.BlockSpec((_EBLK, 6), lambda i: (i, 0)),
                  pl.BlockSpec((_EBLK, 512), lambda i: (i, 0))] + wspec,
        out_specs=pl.BlockSpec((_EBLK, 448), lambda i: (i, 0)),
        out_shape=jax.ShapeDtypeStruct((e, 448), jnp.float32),
        interpret=_INTERPRET,
    )(rpack, gathered, *ws)


def _group_norm0(x, g, b):
    sh = x.shape
    d = sh[-1]
    xg = x.reshape(sh[:-1] + (GROUPS, d // GROUPS))
    mu = xg.mean(-1, keepdims=True)
    var = xg.var(-1, keepdims=True)
    xg = (xg - mu) / jnp.sqrt(var + 1e-5)
    return xg.reshape(sh) * g + b


def _group_norm1(x, g):
    sh = x.shape
    d = sh[-2]
    xg = x.reshape(sh[:-2] + (GROUPS, d // GROUPS, 3))
    rms = jnp.sqrt(jnp.mean(jnp.sum(xg ** 2, -1), -1, keepdims=True) + 1e-5)
    xg = xg / rms[..., None]
    return xg.reshape(sh) * g[:, None]


def _self_tens_prods(p, x_a, x_v):
    l0 = x_a @ p['L0']
    r0 = x_a @ p['R0']
    l1 = _tl1(x_v, p['L1'])
    r1 = _tl1(x_v, p['R1'])
    y_a = (l0 * r0) @ p['Y000'] + jnp.sum(l1 * r1, -1) @ p['Y110']
    y_a = y_a + _mlp(p['mlp'], y_a)
    l0 = l0 + y_a @ p['Ul']
    r0 = r0 + y_a @ p['Ur']
    y_v = (_tl1(l0[..., None] * r1, p['Y011'])
           + _tl1(l1 * r0[..., None], p['Y101'])
           + _tl1(jnp.cross(l1, r1), p['Y111']))
    return _group_norm0(y_a, p['gn_a_g'], p['gn_a_b']), _group_norm1(y_v, p['gn_v_g'])


def kernel(pos_0, pos_1, x_a, x_v, t, atom_idx, edge_index, params):
    src = edge_index[0]
    dst = edge_index[1]
    xa = _prologue(x_a[0], atom_idx[0].astype(jnp.int32), t,
                   params['res_embed'], params['embed_t']['W'],
                   params['embed_t']['b'])
    xv = x_v[0] + (pos_1 - pos_0)[0, :, None, :] * params['pos_embed'][0][:, None]
    n = xa.shape[0]
    rpack = jnp.concatenate([pos_0[0][dst] - pos_0[0][src],
                             pos_1[0][dst] - pos_1[0][src]], axis=1)

    pay = _disptens_edges(rpack, xa[dst], params['disptens_0'], params['disptens_1'])
    A = jax.ops.segment_sum(pay, src, num_segments=N_NODES)
    xa = xa + A[:, :DIM_A] + A[:, 224:224 + DIM_A]
    A_v = (A[:, DIM_A:224].reshape(n, 3, DIM_V)
           + A[:, 224 + DIM_A:].reshape(n, 3, DIM_V)).transpose(0, 2, 1)
    xv = xv + A_v

    y_a, y_v = _self_tens_prods(params['stp'], xa, xv)
    xa = xa + y_a
    xv = xv + y_v

    cols = []
    for p in (params['msg_0'], params['msg_1']):
        cols.append(xa @ p['L0'])
        for k in range(3):
            cols.append(xv[:, :, k] @ p['L1'])
    gathered = jnp.concatenate(cols, axis=1)[dst]

    pay = _messages_edges(rpack, gathered, params['msg_0'], params['msg_1'])
    B = jax.ops.segment_sum(pay, src, num_segments=N_NODES)
    xa = xa + 0.1 * (B[:, :DIM_A] + B[:, 224:224 + DIM_A])
    B_v = (B[:, DIM_A:224].reshape(n, 3, DIM_V)
           + B[:, 224 + DIM_A:].reshape(n, 3, DIM_V)).transpose(0, 2, 1)
    xv = xv + 0.1 * B_v
    return xa[None], xv[None]
